# baseline (speedup 1.0000x reference)
import functools

import jax
import jax.numpy as jnp
from jax import lax
from jax.experimental import pallas as pl
from jax.experimental.pallas import tpu as pltpu
from jax.experimental.pallas import tpu_sc as plsc

B = 16384
MAX_LEN = 50
VOCAB = 1000000
EMBED = 32

NW = 32
HALF_B = B // 2
N_HALF = HALF_B * MAX_LEN
NPW = N_HALF // NW
BCH = 32
CH = BCH * MAX_LEN
NCHUNK = NPW // CH
BPW = HALF_B // NW

_mesh = plsc.VectorSubcoreMesh(core_axis_name="c", subcore_axis_name="s")


def _make_half(half):
    @functools.partial(
        pl.kernel,
        mesh=_mesh,
        compiler_params=pltpu.CompilerParams(use_tc_tiling_on_sc=False),
        out_type=jax.ShapeDtypeStruct((N_HALF, EMBED), jnp.float32),
        scratch_types=[
            pltpu.VMEM((NPW,), jnp.int32),
            pltpu.VMEM((2, CH, EMBED), jnp.float32),
            pltpu.VMEM((BPW + 16,), jnp.int32),
            pltpu.SemaphoreType.DMA,
            pltpu.SemaphoreType.DMA,
            pltpu.SemaphoreType.DMA,
        ],
    )
    def _embed_sc(
        idx_hbm, seq_hbm, table_hbm, out_hbm,
        idx_v, rows_v, seq_v, gsem, osem0, osem1,
    ):
        wid = lax.axis_index("s") * 2 + lax.axis_index("c")
        gbase = half * N_HALF + wid * NPW
        base = wid * NPW
        pltpu.sync_copy(
            seq_hbm.at[pl.ds(half * HALF_B + wid * BPW, BPW)],
            seq_v.at[pl.ds(0, BPW)],
        )
        pltpu.sync_copy(idx_hbm.at[pl.ds(gbase, NPW)], idx_v)
        zvec = jnp.zeros((16,), jnp.float32)
        osems = (osem0, osem1)

        def fire_gather(g):
            return pltpu.async_copy(
                table_hbm.at[idx_v.at[pl.ds(g * CH, CH)]],
                rows_v.at[(g % 2)],
                gsem,
            )

        def zero_chunk(g):
            s = g % 2

            def zero_b(brel, _):
                sl = seq_v[pl.ds(g * BCH + brel, 16)][0]

                def zero_row(r, _):
                    rows_v[s, brel * MAX_LEN + r, pl.ds(0, 16)] = zvec
                    rows_v[s, brel * MAX_LEN + r, pl.ds(16, 16)] = zvec
                    return 0

                lax.fori_loop(sl, MAX_LEN, zero_row, 0)
                return 0

            lax.fori_loop(0, BCH, zero_b, 0)

        out_copies = [None, None]
        gather = fire_gather(0)
        for g in range(NCHUNK):
            s = g % 2
            if g + 1 < NCHUNK:
                if out_copies[1 - s] is not None:
                    out_copies[1 - s].wait()
                next_gather = fire_gather(g + 1)
            gather.wait()
            zero_chunk(g)
            out_copies[s] = pltpu.async_copy(
                rows_v.at[s], out_hbm.at[pl.ds(base + g * CH, CH)], osems[s]
            )
            if g + 1 < NCHUNK:
                gather = next_gather
        for c in out_copies:
            if c is not None:
                c.wait()

    return _embed_sc


_embed_half0 = _make_half(0)
_embed_half1 = _make_half(1)


def kernel(indices, seq_lens, table):
    idx = indices.astype(jnp.int32).reshape(-1)
    seq = seq_lens.astype(jnp.int32)
    out0 = _embed_half0(idx, seq, table)
    out1 = _embed_half1(idx, seq, table)
    return jnp.concatenate(
        [
            out0.reshape(HALF_B, MAX_LEN * EMBED),
            out1.reshape(HALF_B, MAX_LEN * EMBED),
        ],
        axis=0,
    )


# speedup vs baseline: 1.0592x; 1.0592x over previous
import functools

import jax
import jax.numpy as jnp
from jax import lax
from jax.experimental import pallas as pl
from jax.experimental.pallas import tpu as pltpu
from jax.experimental.pallas import tpu_sc as plsc

B = 16384
MAX_LEN = 50
VOCAB = 1000000
EMBED = 32

N = B * MAX_LEN
NW = 32
NPW = N // NW
BCH = 16
CH = BCH * MAX_LEN
NCHUNK = NPW // CH
NBUF = 4
BPW = B // NW

_mesh = plsc.VectorSubcoreMesh(core_axis_name="c", subcore_axis_name="s")


@functools.partial(
    pl.kernel,
    mesh=_mesh,
    compiler_params=pltpu.CompilerParams(use_tc_tiling_on_sc=False),
    out_type=jax.ShapeDtypeStruct((N, EMBED), jnp.float32),
    scratch_types=[
        pltpu.VMEM((NPW,), jnp.int32),
        pltpu.VMEM((NBUF, CH, EMBED), jnp.float32),
        pltpu.VMEM((BPW + 16,), jnp.int32),
        pltpu.SemaphoreType.DMA,
        pltpu.SemaphoreType.DMA,
        pltpu.SemaphoreType.DMA,
        pltpu.SemaphoreType.DMA,
        pltpu.SemaphoreType.DMA,
    ],
)
def _embed_sc(
    idx_hbm, seq_hbm, table_hbm, out_hbm, idx_v, rows_v, seq_v,
    gsem, osem0, osem1, osem2, osem3,
):
    wid = lax.axis_index("s") * 2 + lax.axis_index("c")
    base = wid * NPW
    pltpu.sync_copy(seq_hbm.at[pl.ds(wid * BPW, BPW)], seq_v.at[pl.ds(0, BPW)])
    pltpu.sync_copy(idx_hbm.at[pl.ds(base, NPW)], idx_v)
    zvec = jnp.zeros((16,), jnp.float32)
    osems = (osem0, osem1, osem2, osem3)

    def fire_gather(g):
        return pltpu.async_copy(
            table_hbm.at[idx_v.at[pl.ds(g * CH, CH)]],
            rows_v.at[(g % NBUF)],
            gsem,
        )

    def zero_chunk(g):
        s = g % NBUF

        def zero_b(brel, _):
            sl = seq_v[pl.ds(g * BCH + brel, 16)][0]

            def zero_row(r, _):
                rows_v[s, brel * MAX_LEN + r, pl.ds(0, 16)] = zvec
                rows_v[s, brel * MAX_LEN + r, pl.ds(16, 16)] = zvec
                return 0

            lax.fori_loop(sl, MAX_LEN, zero_row, 0)
            return 0

        lax.fori_loop(0, BCH, zero_b, 0)

    out_copies = [None] * NBUF
    gathers = [None] * NBUF
    for g in range(NBUF - 1):
        gathers[g] = fire_gather(g)
    for g in range(NCHUNK):
        s = g % NBUF
        f = g + NBUF - 1
        if f < NCHUNK:
            sf = f % NBUF
            if out_copies[sf] is not None:
                out_copies[sf].wait()
            gathers[sf] = fire_gather(f)
        gathers[s].wait()
        zero_chunk(g)
        out_copies[s] = pltpu.async_copy(
            rows_v.at[s], out_hbm.at[pl.ds(base + g * CH, CH)], osems[s]
        )
    for c in out_copies:
        if c is not None:
            c.wait()


def kernel(indices, seq_lens, table):
    idx = indices.astype(jnp.int32).reshape(-1)
    seq = seq_lens.astype(jnp.int32)
    out = _embed_sc(idx, seq, table)
    return out.reshape(B, MAX_LEN * EMBED)
